# baseline (speedup 1.0000x reference)
import functools

import jax
import jax.numpy as jnp
from jax import lax
from jax.experimental import pallas as pl
from jax.experimental.pallas import tpu as pltpu
from jax.experimental.pallas import tpu_sc as plsc

N_FACTORS = 16
N_ROWS = 1_000_000
BATCH = 16384
NUM_WORKERS = 32
B_PER_W = BATCH // NUM_WORKERS
CCHUNK = 16
N_CCHUNKS = B_PER_W // CCHUNK
GCHUNK = 128
N_GCHUNKS = B_PER_W // GCHUNK

LANE = 128
TILE_WORDS = 8 * LANE
W_TILES = 7813
TROW_WORDS = W_TILES * TILE_WORDS
FLAT_WORDS = 2 * TROW_WORDS
CHUNK_TILES = 244
EXTRA_TILE0 = 7808
EXTRA_TILES = 4
PAD_TILE = 7812


def _wid():
    return lax.axis_index("s") * 2 + lax.axis_index("c")


DRAIN_LAG = 8


def _detile_body(ut_hbm, it_hbm, tu_hbm, ti_hbm, flat_u, flat_i, sem):
    w = _wid()
    combos = ((ut_hbm, flat_u, 0), (ut_hbm, flat_u, 1),
              (it_hbm, flat_i, 0), (it_hbm, flat_i, 1))
    tails = ((tu_hbm, flat_u, 0), (tu_hbm, flat_u, 1),
             (ti_hbm, flat_i, 0), (ti_hbm, flat_i, 1))

    def tile_copy(src, dst, tr, tc):
        s = src.at[pl.ds(tr * 8, 8),
                   pl.ds(pl.multiple_of(tc * LANE, LANE), LANE)]
        d = dst.at[pl.ds(pl.multiple_of((tr * W_TILES + tc) * 8, 8), 8), :]
        return s, d

    def issue(k):
        tc = pl.multiple_of(w * CHUNK_TILES, CHUNK_TILES) + k
        for (src, dst, tr) in combos:
            s, d = tile_copy(src, dst, tr, tc)
            pltpu.async_copy(s, d, sem)

    def drain(k):
        tc = pl.multiple_of(w * CHUNK_TILES, CHUNK_TILES) + k
        for (src, dst, tr) in combos:
            s, d = tile_copy(src, dst, tr, tc)
            pltpu.make_async_copy(s, d, sem).wait()

    def step(k, _):
        issue(k)

        @pl.when(k >= DRAIN_LAG)
        def _():
            drain(k - DRAIN_LAG)
        return _

    lax.fori_loop(0, CHUNK_TILES, step, None)

    def tail_drain(k, _):
        drain(CHUNK_TILES - DRAIN_LAG + k)
        return _

    lax.fori_loop(0, DRAIN_LAG, tail_drain, None)

    for k, (src, dst, tr) in enumerate(combos):
        @pl.when(w == k)
        def _(src=src, dst=dst, tr=tr):
            for j in range(EXTRA_TILES):
                s, d = tile_copy(src, dst, tr, EXTRA_TILE0 + j)
                pltpu.async_copy(s, d, sem)
            for j in range(EXTRA_TILES):
                s, d = tile_copy(src, dst, tr, EXTRA_TILE0 + j)
                pltpu.make_async_copy(s, d, sem).wait()

    for k, (tsrc, dst, tr) in enumerate(tails):
        @pl.when(w == 4 + k)
        def _(tsrc=tsrc, dst=dst, tr=tr):
            d = dst.at[pl.ds((tr * W_TILES + PAD_TILE) * 8, 8), :]
            pltpu.async_copy(tsrc.at[pl.ds(tr * 8, 8), :], d, sem).wait()


def _gather_body(u_hbm, i_hbm, flat_u, flat_i, out_hbm,
                 idx_u, idx_i, offs_u, offs_i, vals_u, vals_i, out_v, sem):
    w = _wid()
    base = pl.multiple_of(w * B_PER_W, B_PER_W)

    pltpu.sync_copy(u_hbm.at[pl.ds(base, B_PER_W)], idx_u)
    pltpu.sync_copy(i_hbm.at[pl.ds(base, B_PER_W)], idx_i)

    def build(cidx, _):
        csl = pl.ds(pl.multiple_of(cidx * CCHUNK, CCHUNK), CCHUNK)
        for (idx, offs) in ((idx_u, offs_u), (idx_i, offs_i)):
            v = idx[csl]
            pos = ((v >> 7) << 10) + (v & 127)
            for d in range(N_FACTORS):
                offs[d, csl] = pos + ((d % 8) * LANE + (d // 8) * TROW_WORDS)
        return _

    lax.fori_loop(0, N_CCHUNKS, build, None)

    copies = []
    for d in range(N_FACTORS):
        for j in range(N_GCHUNKS):
            sl = pl.ds(j * GCHUNK, GCHUNK)
            copies.append(pltpu.async_copy(
                flat_u.at[offs_u.at[d, sl]], vals_u.at[d, sl], sem))
            copies.append(pltpu.async_copy(
                flat_i.at[offs_i.at[d, sl]], vals_i.at[d, sl], sem))
    for c in copies:
        c.wait()

    def compute(cidx, _):
        csl = pl.ds(pl.multiple_of(cidx * CCHUNK, CCHUNK), CCHUNK)
        acc = jnp.zeros((CCHUNK,), jnp.float32)
        for d in range(N_FACTORS):
            acc = acc + vals_u[d, csl] * vals_i[d, csl]
        out_v[csl] = acc
        return _

    lax.fori_loop(0, N_CCHUNKS, compute, None)

    pltpu.sync_copy(out_v, out_hbm.at[pl.ds(base, B_PER_W)])


@jax.jit
def kernel(u, i, user_emb, item_emb):
    mesh = plsc.VectorSubcoreMesh(core_axis_name="c", subcore_axis_name="s")
    ut = user_emb.T
    it = item_emb.T
    pad = ((0, 0), (0, LANE - (N_ROWS - PAD_TILE * LANE)))
    tu = jnp.pad(ut[:, PAD_TILE * LANE:], pad)
    ti = jnp.pad(it[:, PAD_TILE * LANE:], pad)

    flat = jax.ShapeDtypeStruct((FLAT_WORDS // LANE, LANE), jnp.float32)
    detile = pl.kernel(
        _detile_body,
        mesh=mesh,
        out_type=(flat, flat),
        scratch_types=[pltpu.SemaphoreType.DMA],
        compiler_params=pltpu.CompilerParams(needs_layout_passes=False),
    )
    flat_u2, flat_i2 = detile(ut, it, tu, ti)
    flat_u = flat_u2.reshape(FLAT_WORDS)
    flat_i = flat_i2.reshape(FLAT_WORDS)

    gather = pl.kernel(
        _gather_body,
        mesh=mesh,
        out_type=jax.ShapeDtypeStruct((BATCH,), jnp.float32),
        scratch_types=[
            pltpu.VMEM((B_PER_W,), jnp.int32),
            pltpu.VMEM((B_PER_W,), jnp.int32),
            pltpu.VMEM((N_FACTORS, B_PER_W), jnp.int32),
            pltpu.VMEM((N_FACTORS, B_PER_W), jnp.int32),
            pltpu.VMEM((N_FACTORS, B_PER_W), jnp.float32),
            pltpu.VMEM((N_FACTORS, B_PER_W), jnp.float32),
            pltpu.VMEM((B_PER_W,), jnp.float32),
            pltpu.SemaphoreType.DMA,
        ],
        compiler_params=pltpu.CompilerParams(needs_layout_passes=False),
    )
    return gather(u, i, flat_u, flat_i)


# speedup vs baseline: 20.9109x; 20.9109x over previous
import functools

import jax
import jax.numpy as jnp
from jax import lax
from jax.experimental import pallas as pl
from jax.experimental.pallas import tpu as pltpu
from jax.experimental.pallas import tpu_sc as plsc

N_FACTORS = 16
N_ROWS = 1_000_000
BATCH = 16384
NUM_WORKERS = 32
B_PER_W = BATCH // NUM_WORKERS
CCHUNK = 16
N_CCHUNKS = B_PER_W // CCHUNK
GCHUNK = 128
N_GCHUNKS = B_PER_W // GCHUNK

LANE = 128
TILE_WORDS = 8 * LANE
W_TILES = 7813
TROW_WORDS = W_TILES * TILE_WORDS
FLAT_WORDS = 2 * TROW_WORDS
CHUNK_TILES = 244
EXTRA_TILE0 = 7808
EXTRA_TILES = 4
PAD_TILE = 7812


def _wid():
    return lax.axis_index("s") * 2 + lax.axis_index("c")


SLAB_TILES = 61
N_BATCHES = CHUNK_TILES // SLAB_TILES


def _detile_body(ut_hbm, it_hbm, tu_hbm, ti_hbm, flat_u, flat_i,
                 slab_a, slab_b, sem):
    w = _wid()
    combos = ((ut_hbm, flat_u, 0), (ut_hbm, flat_u, 1),
              (it_hbm, flat_i, 0), (it_hbm, flat_i, 1))
    tails = ((tu_hbm, flat_u, 0), (tu_hbm, flat_u, 1),
             (ti_hbm, flat_i, 0), (ti_hbm, flat_i, 1))

    def relabel(ntiles):
        def one_tile(tc, _):
            r0 = pl.multiple_of(tc * 8, 8)
            c0 = pl.multiple_of(tc * LANE, LANE)
            for d in range(8):
                for p in range(8):
                    slab_b[r0 + d, pl.ds(p * 16, 16)] = (
                        slab_a[d, pl.ds(c0 + p * 16, 16)])
            return _

        lax.fori_loop(0, ntiles, one_tile, None)

    def do_batch(src, dst, tr, tile0, ntiles):
        pltpu.sync_copy(
            src.at[pl.ds(tr * 8, 8),
                   pl.ds(pl.multiple_of(tile0 * LANE, LANE), ntiles * LANE)],
            slab_a.at[:, pl.ds(0, ntiles * LANE)])
        relabel(ntiles)
        pltpu.sync_copy(
            slab_b.at[pl.ds(0, ntiles * 8), :],
            dst.at[pl.ds(pl.multiple_of((tr * W_TILES + tile0) * 8, 8),
                         ntiles * 8), :])

    for (src, dst, tr) in combos:
        for b in range(N_BATCHES):
            do_batch(src, dst, tr,
                     w * CHUNK_TILES + b * SLAB_TILES, SLAB_TILES)

    for k, (src, dst, tr) in enumerate(combos):
        @pl.when(w == k)
        def _(src=src, dst=dst, tr=tr):
            do_batch(src, dst, tr, EXTRA_TILE0, EXTRA_TILES)

    for k, (tsrc, dst, tr) in enumerate(tails):
        @pl.when(w == 4 + k)
        def _(tsrc=tsrc, dst=dst, tr=tr):
            pltpu.sync_copy(tsrc.at[pl.ds(tr * 8, 8), :],
                            slab_a.at[:, pl.ds(0, LANE)])
            relabel(1)
            pltpu.sync_copy(
                slab_b.at[pl.ds(0, 8), :],
                dst.at[pl.ds((tr * W_TILES + PAD_TILE) * 8, 8), :])


def _gather_body(u_hbm, i_hbm, flat_u, flat_i, out_hbm,
                 idx_u, idx_i, offs_u, offs_i, vals_u, vals_i, out_v, sem):
    w = _wid()
    base = pl.multiple_of(w * B_PER_W, B_PER_W)

    pltpu.sync_copy(u_hbm.at[pl.ds(base, B_PER_W)], idx_u)
    pltpu.sync_copy(i_hbm.at[pl.ds(base, B_PER_W)], idx_i)

    def build(cidx, _):
        csl = pl.ds(pl.multiple_of(cidx * CCHUNK, CCHUNK), CCHUNK)
        for (idx, offs) in ((idx_u, offs_u), (idx_i, offs_i)):
            v = idx[csl]
            pos = ((v >> 7) << 10) + (v & 127)
            for d in range(N_FACTORS):
                offs[d, csl] = pos + ((d % 8) * LANE + (d // 8) * TROW_WORDS)
        return _

    lax.fori_loop(0, N_CCHUNKS, build, None)

    copies = []
    for d in range(N_FACTORS):
        for j in range(N_GCHUNKS):
            sl = pl.ds(j * GCHUNK, GCHUNK)
            copies.append(pltpu.async_copy(
                flat_u.at[offs_u.at[d, sl]], vals_u.at[d, sl], sem))
            copies.append(pltpu.async_copy(
                flat_i.at[offs_i.at[d, sl]], vals_i.at[d, sl], sem))
    for c in copies:
        c.wait()

    def compute(cidx, _):
        csl = pl.ds(pl.multiple_of(cidx * CCHUNK, CCHUNK), CCHUNK)
        acc = jnp.zeros((CCHUNK,), jnp.float32)
        for d in range(N_FACTORS):
            acc = acc + vals_u[d, csl] * vals_i[d, csl]
        out_v[csl] = acc
        return _

    lax.fori_loop(0, N_CCHUNKS, compute, None)

    pltpu.sync_copy(out_v, out_hbm.at[pl.ds(base, B_PER_W)])


@jax.jit
def kernel(u, i, user_emb, item_emb):
    mesh = plsc.VectorSubcoreMesh(core_axis_name="c", subcore_axis_name="s")
    ut = user_emb.T
    it = item_emb.T
    pad = ((0, 0), (0, LANE - (N_ROWS - PAD_TILE * LANE)))
    tu = jnp.pad(ut[:, PAD_TILE * LANE:], pad)
    ti = jnp.pad(it[:, PAD_TILE * LANE:], pad)

    flat = jax.ShapeDtypeStruct((FLAT_WORDS // LANE, LANE), jnp.float32)
    detile = pl.kernel(
        _detile_body,
        mesh=mesh,
        out_type=(flat, flat),
        scratch_types=[
            pltpu.VMEM((8, SLAB_TILES * LANE), jnp.float32),
            pltpu.VMEM((SLAB_TILES * 8, LANE), jnp.float32),
            pltpu.SemaphoreType.DMA,
        ],
        compiler_params=pltpu.CompilerParams(needs_layout_passes=False),
    )
    flat_u2, flat_i2 = detile(ut, it, tu, ti)
    flat_u = flat_u2.reshape(FLAT_WORDS)
    flat_i = flat_i2.reshape(FLAT_WORDS)

    gather = pl.kernel(
        _gather_body,
        mesh=mesh,
        out_type=jax.ShapeDtypeStruct((BATCH,), jnp.float32),
        scratch_types=[
            pltpu.VMEM((B_PER_W,), jnp.int32),
            pltpu.VMEM((B_PER_W,), jnp.int32),
            pltpu.VMEM((N_FACTORS, B_PER_W), jnp.int32),
            pltpu.VMEM((N_FACTORS, B_PER_W), jnp.int32),
            pltpu.VMEM((N_FACTORS, B_PER_W), jnp.float32),
            pltpu.VMEM((N_FACTORS, B_PER_W), jnp.float32),
            pltpu.VMEM((B_PER_W,), jnp.float32),
            pltpu.SemaphoreType.DMA,
        ],
        compiler_params=pltpu.CompilerParams(needs_layout_passes=False),
    )
    return gather(u, i, flat_u, flat_i)


# speedup vs baseline: 26.6003x; 1.2721x over previous
import functools

import jax
import jax.numpy as jnp
from jax import lax
from jax.experimental import pallas as pl
from jax.experimental.pallas import tpu as pltpu
from jax.experimental.pallas import tpu_sc as plsc

N_FACTORS = 16
N_ROWS = 1_000_000
BATCH = 16384
NUM_WORKERS = 32
B_PER_W = BATCH // NUM_WORKERS
CCHUNK = 16
N_CCHUNKS = B_PER_W // CCHUNK
GCHUNK = 128
N_GCHUNKS = B_PER_W // GCHUNK

LANE = 128
TILE_WORDS = 8 * LANE
W_TILES = 7813
TROW_WORDS = W_TILES * TILE_WORDS
FLAT_WORDS = 2 * TROW_WORDS
CHUNK_TILES = 244
EXTRA_TILE0 = 7808
EXTRA_TILES = 4
PAD_TILE = 7812


def _wid():
    return lax.axis_index("s") * 2 + lax.axis_index("c")


SLAB_TILES = 40
N_BATCHES = CHUNK_TILES // SLAB_TILES
REM_TILES = CHUNK_TILES - N_BATCHES * SLAB_TILES


def _detile_body(ut_hbm, it_hbm, tu_hbm, ti_hbm, flat_u, flat_i,
                 slab_a0, slab_a1, slab_b, sem_in0, sem_in1, sem_out):
    w = _wid()
    combos = ((ut_hbm, flat_u, 0), (ut_hbm, flat_u, 1),
              (it_hbm, flat_i, 0), (it_hbm, flat_i, 1))
    tails = ((tu_hbm, flat_u, 0), (tu_hbm, flat_u, 1),
             (ti_hbm, flat_i, 0), (ti_hbm, flat_i, 1))
    slabs = (slab_a0, slab_a1)
    sems = (sem_in0, sem_in1)

    def relabel(slab_a, ntiles):
        def one_tile(tc, _):
            r0 = pl.multiple_of(tc * 8, 8)
            c0 = pl.multiple_of(tc * LANE, LANE)
            for d in range(8):
                for p in range(8):
                    slab_b[r0 + d, pl.ds(p * 16, 16)] = (
                        slab_a[d, pl.ds(c0 + p * 16, 16)])
            return _

        lax.fori_loop(0, ntiles, one_tile, None)

    jobs = []
    for (src, dst, tr) in combos:
        for b in range(N_BATCHES):
            jobs.append((src, dst, tr, b * SLAB_TILES, SLAB_TILES))
        jobs.append((src, dst, tr, N_BATCHES * SLAB_TILES, REM_TILES))

    def in_pair(k):
        src, _, tr, t0, n = jobs[k]
        return (
            src.at[pl.ds(tr * 8, 8),
                   pl.ds(pl.multiple_of((w * CHUNK_TILES + t0) * LANE, LANE),
                         n * LANE)],
            slabs[k % 2].at[:, pl.ds(0, n * LANE)],
            sems[k % 2],
        )

    def out_pair(k):
        _, dst, tr, t0, n = jobs[k]
        r0 = (tr * W_TILES + w * CHUNK_TILES + t0) * 8
        return (
            slab_b.at[pl.ds(0, n * 8), :],
            dst.at[pl.ds(pl.multiple_of(r0, 8), n * 8), :],
            sem_out,
        )

    pltpu.async_copy(*in_pair(0))
    for k, job in enumerate(jobs):
        pltpu.make_async_copy(*in_pair(k)).wait()
        if k + 1 < len(jobs):
            pltpu.async_copy(*in_pair(k + 1))
        if k > 0:
            pltpu.make_async_copy(*out_pair(k - 1)).wait()
        relabel(slabs[k % 2], job[4])
        pltpu.async_copy(*out_pair(k))
    pltpu.make_async_copy(*out_pair(len(jobs) - 1)).wait()

    def sync_batch(src_ref, dst, tr, dst_tile0, ntiles):
        pltpu.sync_copy(src_ref, slab_a0.at[:, pl.ds(0, ntiles * LANE)])
        relabel(slab_a0, ntiles)
        pltpu.sync_copy(
            slab_b.at[pl.ds(0, ntiles * 8), :],
            dst.at[pl.ds((tr * W_TILES + dst_tile0) * 8, ntiles * 8), :])

    for k, (src, dst, tr) in enumerate(combos):
        @pl.when(w == k)
        def _(src=src, dst=dst, tr=tr):
            sync_batch(
                src.at[pl.ds(tr * 8, 8),
                       pl.ds(EXTRA_TILE0 * LANE, EXTRA_TILES * LANE)],
                dst, tr, EXTRA_TILE0, EXTRA_TILES)

    for k, (tsrc, dst, tr) in enumerate(tails):
        @pl.when(w == 4 + k)
        def _(tsrc=tsrc, dst=dst, tr=tr):
            sync_batch(tsrc.at[pl.ds(tr * 8, 8), :], dst, tr, PAD_TILE, 1)


def _gather_body(u_hbm, i_hbm, flat_u, flat_i, out_hbm,
                 idx_u, idx_i, offs_u, offs_i, vals_u, vals_i, out_v, sem):
    w = _wid()
    base = pl.multiple_of(w * B_PER_W, B_PER_W)

    pltpu.sync_copy(u_hbm.at[pl.ds(base, B_PER_W)], idx_u)
    pltpu.sync_copy(i_hbm.at[pl.ds(base, B_PER_W)], idx_i)

    def build(cidx, _):
        csl = pl.ds(pl.multiple_of(cidx * CCHUNK, CCHUNK), CCHUNK)
        for (idx, offs) in ((idx_u, offs_u), (idx_i, offs_i)):
            v = idx[csl]
            pos = ((v >> 7) << 10) + (v & 127)
            for d in range(N_FACTORS):
                offs[d, csl] = pos + ((d % 8) * LANE + (d // 8) * TROW_WORDS)
        return _

    lax.fori_loop(0, N_CCHUNKS, build, None)

    copies = []
    for d in range(N_FACTORS):
        for j in range(N_GCHUNKS):
            sl = pl.ds(j * GCHUNK, GCHUNK)
            copies.append(pltpu.async_copy(
                flat_u.at[offs_u.at[d, sl]], vals_u.at[d, sl], sem))
            copies.append(pltpu.async_copy(
                flat_i.at[offs_i.at[d, sl]], vals_i.at[d, sl], sem))
    for c in copies:
        c.wait()

    def compute(cidx, _):
        csl = pl.ds(pl.multiple_of(cidx * CCHUNK, CCHUNK), CCHUNK)
        acc = jnp.zeros((CCHUNK,), jnp.float32)
        for d in range(N_FACTORS):
            acc = acc + vals_u[d, csl] * vals_i[d, csl]
        out_v[csl] = acc
        return _

    lax.fori_loop(0, N_CCHUNKS, compute, None)

    pltpu.sync_copy(out_v, out_hbm.at[pl.ds(base, B_PER_W)])


@jax.jit
def kernel(u, i, user_emb, item_emb):
    mesh = plsc.VectorSubcoreMesh(core_axis_name="c", subcore_axis_name="s")
    ut = user_emb.T
    it = item_emb.T
    pad = ((0, 0), (0, LANE - (N_ROWS - PAD_TILE * LANE)))
    tu = jnp.pad(ut[:, PAD_TILE * LANE:], pad)
    ti = jnp.pad(it[:, PAD_TILE * LANE:], pad)

    flat = jax.ShapeDtypeStruct((FLAT_WORDS // LANE, LANE), jnp.float32)
    detile = pl.kernel(
        _detile_body,
        mesh=mesh,
        out_type=(flat, flat),
        scratch_types=[
            pltpu.VMEM((8, SLAB_TILES * LANE), jnp.float32),
            pltpu.VMEM((8, SLAB_TILES * LANE), jnp.float32),
            pltpu.VMEM((SLAB_TILES * 8, LANE), jnp.float32),
            pltpu.SemaphoreType.DMA,
            pltpu.SemaphoreType.DMA,
            pltpu.SemaphoreType.DMA,
        ],
        compiler_params=pltpu.CompilerParams(needs_layout_passes=False),
    )
    flat_u2, flat_i2 = detile(ut, it, tu, ti)
    flat_u = flat_u2.reshape(FLAT_WORDS)
    flat_i = flat_i2.reshape(FLAT_WORDS)

    gather = pl.kernel(
        _gather_body,
        mesh=mesh,
        out_type=jax.ShapeDtypeStruct((BATCH,), jnp.float32),
        scratch_types=[
            pltpu.VMEM((B_PER_W,), jnp.int32),
            pltpu.VMEM((B_PER_W,), jnp.int32),
            pltpu.VMEM((N_FACTORS, B_PER_W), jnp.int32),
            pltpu.VMEM((N_FACTORS, B_PER_W), jnp.int32),
            pltpu.VMEM((N_FACTORS, B_PER_W), jnp.float32),
            pltpu.VMEM((N_FACTORS, B_PER_W), jnp.float32),
            pltpu.VMEM((B_PER_W,), jnp.float32),
            pltpu.SemaphoreType.DMA,
        ],
        compiler_params=pltpu.CompilerParams(needs_layout_passes=False),
    )
    return gather(u, i, flat_u, flat_i)
